# Initial kernel scaffold; baseline (speedup 1.0000x reference)
import jax
import jax.numpy as jnp
from jax.experimental import pallas as pl


def kernel(lut, tdata):
    raise NotImplementedError("write your pallas kernel here")


# speedup vs baseline: 1.4379x; 1.4379x over previous
import functools

import jax
import jax.numpy as jnp
from jax import lax
from jax.experimental import pallas as pl
from jax.experimental.pallas import tpu as pltpu
from jax.experimental.pallas import tpu_sc as plsc

_NC = 2
_NS = 16
_NW = _NC * _NS
_CH = 128


@functools.lru_cache(maxsize=None)
def _build(n_rows: int, d: int, n_idx: int):
    assert n_idx % (_NW * _CH) == 0
    per_w = n_idx // _NW
    n_ch = per_w // _CH

    mesh = plsc.VectorSubcoreMesh(core_axis_name="c", subcore_axis_name="s")

    @functools.partial(
        pl.kernel,
        mesh=mesh,
        compiler_params=pltpu.CompilerParams(use_tc_tiling_on_sc=False),
        out_type=jax.ShapeDtypeStruct((n_idx, d), jnp.float32),
        scratch_types=[
            pltpu.VMEM((n_ch, _CH), jnp.int32),
            pltpu.VMEM((_CH, d), jnp.float32),
            pltpu.SemaphoreType.DMA,
        ],
    )
    def gather_kernel(lut_hbm, idx_hbm, out_hbm, idx_v, rows_v, sem):
        wid = lax.axis_index("s") * _NC + lax.axis_index("c")
        base = wid * per_w
        pltpu.sync_copy(idx_hbm.at[wid], idx_v)

        def body(j, carry):
            pltpu.async_copy(lut_hbm.at[idx_v.at[j]], rows_v, sem).wait()
            pltpu.sync_copy(rows_v, out_hbm.at[pl.ds(base + j * _CH, _CH)])
            return carry

        lax.fori_loop(0, n_ch, body, 0)

    return gather_kernel


def kernel(lut, tdata):
    n_rows, d = lut.shape
    b0, b1 = tdata.shape
    n_idx = b0 * b1
    idx = tdata.astype(jnp.int32).reshape(_NW, n_idx // (_NW * _CH), _CH)
    out = _build(n_rows, d, n_idx)(lut, idx)
    return out.reshape(b0, b1, d)


# speedup vs baseline: 1.5763x; 1.0962x over previous
import functools

import jax
import jax.numpy as jnp
from jax import lax
from jax.experimental import pallas as pl
from jax.experimental.pallas import tpu as pltpu
from jax.experimental.pallas import tpu_sc as plsc

_NC = 2
_NS = 16
_NW = _NC * _NS
_CH = 128


_K = 8


@functools.lru_cache(maxsize=None)
def _build(n_rows: int, d: int, n_idx: int):
    assert n_idx % (_NW * _CH * _K) == 0
    per_w = n_idx // _NW
    n_ch = per_w // _CH
    n_g = n_ch // _K
    grp = _K * _CH

    mesh = plsc.VectorSubcoreMesh(core_axis_name="c", subcore_axis_name="s")

    @functools.partial(
        pl.kernel,
        mesh=mesh,
        compiler_params=pltpu.CompilerParams(use_tc_tiling_on_sc=False),
        out_type=jax.ShapeDtypeStruct((n_idx, d), jnp.float32),
        scratch_types=[
            pltpu.VMEM((n_ch, _CH), jnp.int32),
            pltpu.VMEM((2, grp, d), jnp.float32),
            pltpu.SemaphoreType.DMA((2,)),
            pltpu.SemaphoreType.DMA((2,)),
        ],
    )
    def gather_kernel(lut_hbm, idx_hbm, out_hbm, idx_v, buf_v, gsem, wsem):
        wid = lax.axis_index("s") * _NC + lax.axis_index("c")
        base = wid * per_w
        pltpu.sync_copy(idx_hbm.at[wid], idx_v)

        def fire_group(g, b):
            for k in range(_K):
                pltpu.async_copy(
                    lut_hbm.at[idx_v.at[g * _K + k]],
                    buf_v.at[b].at[pl.ds(k * _CH, _CH)],
                    gsem.at[b],
                )

        def drain(sem_ref, b, g):
            pltpu.make_async_copy(
                out_hbm.at[pl.ds(base + g * grp, grp)], buf_v.at[b], sem_ref.at[b]
            ).wait()

        fire_group(0, 0)

        def body(g, carry):
            b = lax.rem(g, 2)
            nb = 1 - b

            @pl.when(g + 1 < n_g)
            def _():
                @pl.when(g >= 1)
                def _():
                    drain(wsem, nb, g - 1)

                fire_group(g + 1, nb)

            drain(gsem, b, g)
            pltpu.async_copy(
                buf_v.at[b], out_hbm.at[pl.ds(base + g * grp, grp)], wsem.at[b]
            )
            return carry

        lax.fori_loop(0, n_g, body, 0)
        drain(wsem, (n_g - 2) % 2, n_g - 2)
        drain(wsem, (n_g - 1) % 2, n_g - 1)

    return gather_kernel


def kernel(lut, tdata):
    n_rows, d = lut.shape
    b0, b1 = tdata.shape
    n_idx = b0 * b1
    idx = tdata.astype(jnp.int32).reshape(_NW, n_idx // (_NW * _CH), _CH)
    out = _build(n_rows, d, n_idx)(lut, idx)
    return out.reshape(b0, b1, d)
